# Initial kernel scaffold; baseline (speedup 1.0000x reference)
import jax
import jax.numpy as jnp
from jax.experimental import pallas as pl


def kernel(x, edge_index, edge_attr, W1, b1, W2, b2):
    raise NotImplementedError("write your pallas kernel here")


# speedup vs baseline: 4.2681x; 4.2681x over previous
import functools

import jax
import jax.numpy as jnp
from jax import lax
from jax.experimental import pallas as pl
from jax.experimental.pallas import tpu as pltpu
from jax.experimental.pallas import tpu_sc as plsc

NN = 10000
NE = 320000
DF = 128
DE = 16
H = 32

NC, NS, L = 2, 16, 16
NW = NC * NS
EPW = NE // NW
CH = 80
NCH = EPW // CH
NP = 10240
ZR = NP // NS


def _node_proj(x, W1a, W1b):
    def body(x_ref, wa_ref, wb_ref, pa_ref, pb_ref):
        xv = x_ref[...]
        pa_ref[...] = jnp.dot(xv, wa_ref[...], preferred_element_type=jnp.float32)
        pb_ref[...] = jnp.dot(xv, wb_ref[...], preferred_element_type=jnp.float32)

    return pl.pallas_call(
        body,
        out_shape=[jax.ShapeDtypeStruct((NN, H), jnp.float32)] * 2,
    )(x, W1a, W1b)


def _edge_proj(edge_attr, W1c, b1):
    BLK = 8000

    def body(ea_ref, w_ref, b_ref, out_ref):
        out_ref[...] = (
            jnp.dot(ea_ref[...], w_ref[...], preferred_element_type=jnp.float32)
            + b_ref[...]
        )

    return pl.pallas_call(
        body,
        grid=(NE // BLK,),
        in_specs=[
            pl.BlockSpec((BLK, DE), lambda i: (i, 0)),
            pl.BlockSpec((DE, H), lambda i: (0, 0)),
            pl.BlockSpec((1, H), lambda i: (0, 0)),
        ],
        out_specs=pl.BlockSpec((BLK, H), lambda i: (i, 0)),
        out_shape=jax.ShapeDtypeStruct((NE, H), jnp.float32),
    )(edge_attr, W1c, b1.reshape(1, H))


def _sc_aggregate(idx_i, idx_j, pa, pb, ew):
    mesh = plsc.VectorSubcoreMesh(
        core_axis_name="c", subcore_axis_name="s", num_cores=NC, num_subcores=NS
    )

    @functools.partial(
        pl.kernel,
        out_type=[
            jax.ShapeDtypeStruct((NC, NP, H), jnp.float32),
            jax.ShapeDtypeStruct((NC, NP), jnp.float32),
        ],
        mesh=mesh,
        compiler_params=pltpu.CompilerParams(use_tc_tiling_on_sc=False),
        scratch_types=[
            pltpu.VMEM((CH,), jnp.int32),
            pltpu.VMEM((CH,), jnp.int32),
            pltpu.VMEM((CH, H), jnp.float32),
            pltpu.VMEM((CH, H), jnp.float32),
            pltpu.VMEM((CH, H), jnp.float32),
            pltpu.VMEM((CH, H), jnp.float32),
            pltpu.VMEM((CH,), jnp.float32),
            pltpu.VMEM((ZR, H), jnp.float32),
            pltpu.VMEM((ZR,), jnp.float32),
            pltpu.VMEM_SHARED((NP, H), jnp.float32),
            pltpu.VMEM_SHARED((NP,), jnp.float32),
            pltpu.SemaphoreType.DMA,
            pltpu.SemaphoreType.DMA,
            pltpu.SemaphoreType.DMA,
        ],
    )
    def k(ii_h, jj_h, pa_h, pb_h, ew_h, out_h, cnt_h,
          ii_v, jj_v, pa_v, pb_v, ew_v, h_v, ones_v, z_v, zc_v,
          acc_sh, cnt_sh, sem1, sem2, sem3):
        cid = lax.axis_index("c")
        sid = lax.axis_index("s")
        wid = cid * NS + sid

        zeros16 = jnp.zeros((L,), jnp.float32)
        ones16 = jnp.ones((L,), jnp.float32)

        def zrow(r, _):
            z_v[r, pl.ds(0, L)] = zeros16
            z_v[r, pl.ds(L, L)] = zeros16
            return 0

        lax.fori_loop(0, ZR, zrow, 0)

        def zcrow(r, _):
            zc_v[pl.ds(r * L, L)] = zeros16
            return 0

        lax.fori_loop(0, ZR // L, zcrow, 0)

        def orow(r, _):
            ones_v[pl.ds(r * L, L)] = ones16
            return 0

        lax.fori_loop(0, CH // L, orow, 0)

        pltpu.sync_copy(z_v, acc_sh.at[pl.ds(sid * ZR, ZR)])
        pltpu.sync_copy(zc_v, cnt_sh.at[pl.ds(sid * ZR, ZR)])
        plsc.subcore_barrier()

        def chunk(c, _):
            base = wid * EPW + c * CH
            pltpu.sync_copy(ii_h.at[pl.ds(base, CH)], ii_v)
            pltpu.sync_copy(jj_h.at[pl.ds(base, CH)], jj_v)
            d1 = pltpu.async_copy(pa_h.at[ii_v], pa_v, sem1)
            d2 = pltpu.async_copy(pb_h.at[jj_v], pb_v, sem2)
            d3 = pltpu.async_copy(ew_h.at[pl.ds(base, CH)], ew_v, sem3)
            d1.wait()
            d2.wait()
            d3.wait()

            def edge(e, _):
                a0 = pa_v[e, pl.ds(0, L)]
                b0 = pb_v[e, pl.ds(0, L)]
                e0 = ew_v[e, pl.ds(0, L)]
                h_v[e, pl.ds(0, L)] = jnp.maximum(a0 + b0 + e0, 0.0)
                a1 = pa_v[e, pl.ds(L, L)]
                b1_ = pb_v[e, pl.ds(L, L)]
                e1 = ew_v[e, pl.ds(L, L)]
                h_v[e, pl.ds(L, L)] = jnp.maximum(a1 + b1_ + e1, 0.0)
                return 0

            lax.fori_loop(0, CH, edge, 0)
            pltpu.sync_copy(h_v, acc_sh.at[ii_v], add=True)
            pltpu.sync_copy(ones_v, cnt_sh.at[ii_v], add=True)
            return 0

        lax.fori_loop(0, NCH, chunk, 0)
        plsc.subcore_barrier()
        pltpu.sync_copy(
            acc_sh.at[pl.ds(sid * ZR, ZR)], out_h.at[cid, pl.ds(sid * ZR, ZR)]
        )
        pltpu.sync_copy(
            cnt_sh.at[pl.ds(sid * ZR, ZR)], cnt_h.at[cid, pl.ds(sid * ZR, ZR)]
        )

    return k(idx_i, idx_j, pa, pb, ew)


def _finalize(parts, cnts, W2, b2):
    def body(s_ref, c_ref, w_ref, b_ref, out_ref):
        s = s_ref[0] + s_ref[1]
        c = c_ref[0] + c_ref[1]
        out_ref[...] = (
            jnp.dot(s[:NN], w_ref[...], preferred_element_type=jnp.float32)
            + c[:NN] * b_ref[...]
        )

    return pl.pallas_call(
        body,
        out_shape=jax.ShapeDtypeStruct((NN, H), jnp.float32),
    )(parts, cnts, W2, b2.reshape(1, H))


def kernel(x, edge_index, edge_attr, W1, b1, W2, b2):
    idx_i = edge_index[0]
    idx_j = edge_index[1]
    W1a = W1[:DF]
    W1b = W1[DF:2 * DF]
    W1c = W1[2 * DF:]
    pa, pb = _node_proj(x, W1a, W1b)
    ew = _edge_proj(edge_attr, W1c, b1)
    parts, cnts = _sc_aggregate(idx_i, idx_j, pa, pb, ew)
    cnts = cnts.reshape(NC, NP, 1)
    return _finalize(parts, cnts, W2, b2)


# speedup vs baseline: 5.4061x; 1.2666x over previous
import functools

import jax
import jax.numpy as jnp
from jax import lax
from jax.experimental import pallas as pl
from jax.experimental.pallas import tpu as pltpu
from jax.experimental.pallas import tpu_sc as plsc

NN = 10000
NE = 320000
DF = 128
DE = 16
H = 32

NC, NS, L = 2, 16, 16
NW = NC * NS
EPW = NE // NW
CH = 40
NCH = EPW // CH
NP = 10240
ZR = NP // NS


def _node_proj(x, W1a, W1b):
    def body(x_ref, wa_ref, wb_ref, pa_ref, pb_ref):
        xv = x_ref[...]
        pa_ref[...] = jnp.dot(xv, wa_ref[...], preferred_element_type=jnp.float32)
        pb_ref[...] = jnp.dot(xv, wb_ref[...], preferred_element_type=jnp.float32)

    return pl.pallas_call(
        body,
        out_shape=[jax.ShapeDtypeStruct((NN, H), jnp.float32)] * 2,
    )(x, W1a, W1b)


def _edge_proj(edge_attr, W1c, b1):
    BLK = 8000

    def body(ea_ref, w_ref, b_ref, out_ref):
        out_ref[...] = (
            jnp.dot(ea_ref[...], w_ref[...], preferred_element_type=jnp.float32)
            + b_ref[...]
        )

    return pl.pallas_call(
        body,
        grid=(NE // BLK,),
        in_specs=[
            pl.BlockSpec((BLK, DE), lambda i: (i, 0)),
            pl.BlockSpec((DE, H), lambda i: (0, 0)),
            pl.BlockSpec((1, H), lambda i: (0, 0)),
        ],
        out_specs=pl.BlockSpec((BLK, H), lambda i: (i, 0)),
        out_shape=jax.ShapeDtypeStruct((NE, H), jnp.float32),
    )(edge_attr, W1c, b1.reshape(1, H))


def _sc_aggregate(idx_i, idx_j, pa, pb, ew):
    mesh = plsc.VectorSubcoreMesh(
        core_axis_name="c", subcore_axis_name="s", num_cores=NC, num_subcores=NS
    )

    @functools.partial(
        pl.kernel,
        out_type=[
            jax.ShapeDtypeStruct((NC, NP, H), jnp.float32),
            jax.ShapeDtypeStruct((NC, NP), jnp.float32),
        ],
        mesh=mesh,
        compiler_params=pltpu.CompilerParams(use_tc_tiling_on_sc=False),
        scratch_types=[
            pltpu.VMEM((NCH, CH), jnp.int32),
            pltpu.VMEM((NCH, CH), jnp.int32),
            pltpu.VMEM((2, CH, H), jnp.float32),
            pltpu.VMEM((2, CH, H), jnp.float32),
            pltpu.VMEM((2, CH, H), jnp.float32),
            pltpu.VMEM((2, CH, H), jnp.float32),
            pltpu.VMEM((CH,), jnp.float32),
            pltpu.VMEM((ZR, H), jnp.float32),
            pltpu.VMEM((ZR,), jnp.float32),
            pltpu.VMEM_SHARED((NP, H), jnp.float32),
            pltpu.VMEM_SHARED((NP,), jnp.float32),
            [pltpu.SemaphoreType.DMA] * 2,
            [pltpu.SemaphoreType.DMA] * 2,
            [pltpu.SemaphoreType.DMA] * 2,
        ],
    )
    def k(ii_h, jj_h, pa_h, pb_h, ew_h, out_h, cnt_h,
          ii_all, jj_all, pa_v, pb_v, ew_v, h_v, ones_v, z_v, zc_v,
          acc_sh, cnt_sh, sem_pa, sem_pb, sem_ew):
        cid = lax.axis_index("c")
        sid = lax.axis_index("s")
        wid = cid * NS + sid

        zeros16 = jnp.zeros((L,), jnp.float32)
        ones16 = jnp.ones((L,), jnp.float32)

        def zrow(r, _):
            z_v[r, pl.ds(0, L)] = zeros16
            z_v[r, pl.ds(L, L)] = zeros16
            return 0

        lax.fori_loop(0, ZR, zrow, 0)

        def zcrow(r, _):
            zc_v[pl.ds(r * L, L)] = zeros16
            return 0

        lax.fori_loop(0, ZR // L, zcrow, 0)

        def orow(r, _):
            ones_v[pl.ds(r * L, L)] = ones16
            return 0

        lax.fori_loop(0, max(CH // L, 1), orow, 0)

        pltpu.sync_copy(ii_h.at[pl.ds(wid * NCH, NCH)], ii_all)
        pltpu.sync_copy(jj_h.at[pl.ds(wid * NCH, NCH)], jj_all)

        pltpu.sync_copy(z_v, acc_sh.at[pl.ds(sid * ZR, ZR)])
        pltpu.sync_copy(zc_v, cnt_sh.at[pl.ds(sid * ZR, ZR)])
        plsc.subcore_barrier()

        def issue(b, c):
            base = wid * EPW + c * CH
            pltpu.async_copy(pa_h.at[ii_all.at[c]], pa_v.at[b], sem_pa[b])
            pltpu.async_copy(pb_h.at[jj_all.at[c]], pb_v.at[b], sem_pb[b])
            pltpu.async_copy(ew_h.at[pl.ds(base, CH)], ew_v.at[b], sem_ew[b])

        def process(b, c):
            pltpu.make_async_copy(pa_h.at[ii_all.at[c]], pa_v.at[b], sem_pa[b]).wait()
            pltpu.make_async_copy(pb_h.at[jj_all.at[c]], pb_v.at[b], sem_pb[b]).wait()
            pltpu.make_async_copy(
                ew_h.at[pl.ds(wid * EPW + c * CH, CH)], ew_v.at[b], sem_ew[b]
            ).wait()

            def edge(e, _):
                a0 = pa_v[b, e, pl.ds(0, L)]
                b0 = pb_v[b, e, pl.ds(0, L)]
                e0 = ew_v[b, e, pl.ds(0, L)]
                h_v[b, e, pl.ds(0, L)] = jnp.maximum(a0 + b0 + e0, 0.0)
                a1 = pa_v[b, e, pl.ds(L, L)]
                b1_ = pb_v[b, e, pl.ds(L, L)]
                e1 = ew_v[b, e, pl.ds(L, L)]
                h_v[b, e, pl.ds(L, L)] = jnp.maximum(a1 + b1_ + e1, 0.0)
                return 0

            lax.fori_loop(0, CH, edge, 0)
            pltpu.sync_copy(h_v.at[b], acc_sh.at[ii_all.at[c]], add=True)
            pltpu.sync_copy(ones_v, cnt_sh.at[ii_all.at[c]], add=True)

        issue(0, 0)

        def pair(c2, _):
            ce = 2 * c2
            issue(1, ce + 1)
            process(0, ce)

            @pl.when(ce + 2 < NCH)
            def _():
                issue(0, ce + 2)

            process(1, ce + 1)
            return 0

        lax.fori_loop(0, NCH // 2, pair, 0)
        plsc.subcore_barrier()
        pltpu.sync_copy(
            acc_sh.at[pl.ds(sid * ZR, ZR)], out_h.at[cid, pl.ds(sid * ZR, ZR)]
        )
        pltpu.sync_copy(
            cnt_sh.at[pl.ds(sid * ZR, ZR)], cnt_h.at[cid, pl.ds(sid * ZR, ZR)]
        )

    return k(idx_i, idx_j, pa, pb, ew)


def _finalize(parts, cnts, W2, b2):
    def body(s_ref, c_ref, w_ref, b_ref, out_ref):
        s = s_ref[0] + s_ref[1]
        c = c_ref[0] + c_ref[1]
        out_ref[...] = (
            jnp.dot(s[:NN], w_ref[...], preferred_element_type=jnp.float32)
            + c[:NN] * b_ref[...]
        )

    return pl.pallas_call(
        body,
        out_shape=jax.ShapeDtypeStruct((NN, H), jnp.float32),
    )(parts, cnts, W2, b2.reshape(1, H))


def kernel(x, edge_index, edge_attr, W1, b1, W2, b2):
    idx_i = edge_index[0].reshape(NE // CH, CH)
    idx_j = edge_index[1].reshape(NE // CH, CH)
    W1a = W1[:DF]
    W1b = W1[DF:2 * DF]
    W1c = W1[2 * DF:]
    pa, pb = _node_proj(x, W1a, W1b)
    ew = _edge_proj(edge_attr, W1c, b1)
    parts, cnts = _sc_aggregate(idx_i, idx_j, pa, pb, ew)
    cnts = cnts.reshape(NC, NP, 1)
    return _finalize(parts, cnts, W2, b2)


# speedup vs baseline: 7.1973x; 1.3313x over previous
import functools

import jax
import jax.numpy as jnp
from jax import lax
from jax.experimental import pallas as pl
from jax.experimental.pallas import tpu as pltpu
from jax.experimental.pallas import tpu_sc as plsc

NN = 10000
NE = 320000
DF = 128
DE = 16
H = 32

NC, NS, L = 2, 16, 16
NW = NC * NS
EPW = NE // NW
CH = 40
NCH = EPW // CH
NP = 10240
ZR = NP // NS


def _node_proj(x, W1a, W1b):
    def body(x_ref, wa_ref, wb_ref, pa_ref, pb_ref):
        xv = x_ref[...]
        pa_ref[...] = jnp.dot(xv, wa_ref[...], preferred_element_type=jnp.float32)
        pb_ref[...] = jnp.dot(xv, wb_ref[...], preferred_element_type=jnp.float32)

    return pl.pallas_call(
        body,
        out_shape=[jax.ShapeDtypeStruct((NN, H), jnp.float32)] * 2,
    )(x, W1a, W1b)


def _edge_proj(edge_attr, W1c, b1):
    BLK = 2000

    def body(ea_ref, w_ref, b_ref, out_ref):
        out_ref[...] = (
            jnp.dot(ea_ref[...], w_ref[...], preferred_element_type=jnp.float32)
            + b_ref[...]
        )

    w_blk = jnp.kron(jnp.eye(4, dtype=jnp.float32), W1c)
    b_tile = jnp.tile(b1, 4).reshape(1, 4 * H)
    ea4 = edge_attr.reshape(NE // 4, 4 * DE)
    return pl.pallas_call(
        body,
        grid=(NE // 4 // BLK,),
        in_specs=[
            pl.BlockSpec((BLK, 4 * DE), lambda i: (i, 0)),
            pl.BlockSpec((4 * DE, 4 * H), lambda i: (0, 0)),
            pl.BlockSpec((1, 4 * H), lambda i: (0, 0)),
        ],
        out_specs=pl.BlockSpec((BLK, 4 * H), lambda i: (i, 0)),
        out_shape=jax.ShapeDtypeStruct((NE // 4, 4 * H), jnp.float32),
    )(ea4, w_blk, b_tile)


def _sc_aggregate(idx_i, idx_j, pa, pb, ew):
    mesh = plsc.VectorSubcoreMesh(
        core_axis_name="c", subcore_axis_name="s", num_cores=NC, num_subcores=NS
    )

    @functools.partial(
        pl.kernel,
        out_type=[
            jax.ShapeDtypeStruct((NC, NP, H), jnp.float32),
            jax.ShapeDtypeStruct((NC, NP), jnp.float32),
        ],
        mesh=mesh,
        compiler_params=pltpu.CompilerParams(use_tc_tiling_on_sc=False),
        scratch_types=[
            pltpu.VMEM((NCH, CH), jnp.int32),
            pltpu.VMEM((NCH, CH), jnp.int32),
            pltpu.VMEM((2, CH, H), jnp.float32),
            pltpu.VMEM((2, CH, H), jnp.float32),
            pltpu.VMEM((2, CH // 4, 4 * H), jnp.float32),
            pltpu.VMEM((2, CH, H), jnp.float32),
            pltpu.VMEM((CH,), jnp.float32),
            pltpu.VMEM((ZR, H), jnp.float32),
            pltpu.VMEM((ZR,), jnp.float32),
            pltpu.VMEM_SHARED((NP, H), jnp.float32),
            pltpu.VMEM_SHARED((NP,), jnp.float32),
            [pltpu.SemaphoreType.DMA] * 2,
            [pltpu.SemaphoreType.DMA] * 2,
            [pltpu.SemaphoreType.DMA] * 2,
        ],
    )
    def k(ii_h, jj_h, pa_h, pb_h, ew_h, out_h, cnt_h,
          ii_all, jj_all, pa_v, pb_v, ew_v, h_v, ones_v, z_v, zc_v,
          acc_sh, cnt_sh, sem_pa, sem_pb, sem_ew):
        cid = lax.axis_index("c")
        sid = lax.axis_index("s")
        wid = cid * NS + sid

        zeros16 = jnp.zeros((L,), jnp.float32)
        ones16 = jnp.ones((L,), jnp.float32)

        def zrow(r, _):
            z_v[r, pl.ds(0, L)] = zeros16
            z_v[r, pl.ds(L, L)] = zeros16
            return 0

        lax.fori_loop(0, ZR, zrow, 0)

        def zcrow(r, _):
            zc_v[pl.ds(r * L, L)] = zeros16
            return 0

        lax.fori_loop(0, ZR // L, zcrow, 0)

        def orow(r, _):
            ones_v[pl.ds(r * L, L)] = ones16
            return 0

        lax.fori_loop(0, max(CH // L, 1), orow, 0)

        pltpu.sync_copy(ii_h.at[pl.ds(wid * NCH, NCH)], ii_all)
        pltpu.sync_copy(jj_h.at[pl.ds(wid * NCH, NCH)], jj_all)

        pltpu.sync_copy(z_v, acc_sh.at[pl.ds(sid * ZR, ZR)])
        pltpu.sync_copy(zc_v, cnt_sh.at[pl.ds(sid * ZR, ZR)])
        plsc.subcore_barrier()

        def issue(b, c):
            base4 = (wid * EPW) // 4 + c * (CH // 4)
            pltpu.async_copy(pa_h.at[ii_all.at[c]], pa_v.at[b], sem_pa[b])
            pltpu.async_copy(pb_h.at[jj_all.at[c]], pb_v.at[b], sem_pb[b])
            pltpu.async_copy(ew_h.at[pl.ds(base4, CH // 4)], ew_v.at[b], sem_ew[b])

        def process(b, c):
            pltpu.make_async_copy(pa_h.at[ii_all.at[c]], pa_v.at[b], sem_pa[b]).wait()
            pltpu.make_async_copy(pb_h.at[jj_all.at[c]], pb_v.at[b], sem_pb[b]).wait()
            pltpu.make_async_copy(
                ew_h.at[pl.ds((wid * EPW) // 4 + c * (CH // 4), CH // 4)],
                ew_v.at[b], sem_ew[b],
            ).wait()

            def row(r, _):
                for kk in range(4):
                    e = r * 4 + kk
                    for hh in range(2):
                        a = pa_v[b, e, pl.ds(hh * L, L)]
                        bb = pb_v[b, e, pl.ds(hh * L, L)]
                        ee = ew_v[b, r, pl.ds(kk * H + hh * L, L)]
                        h_v[b, e, pl.ds(hh * L, L)] = jnp.maximum(a + bb + ee, 0.0)
                return 0

            lax.fori_loop(0, CH // 4, row, 0)
            pltpu.sync_copy(h_v.at[b], acc_sh.at[ii_all.at[c]], add=True)
            pltpu.sync_copy(ones_v, cnt_sh.at[ii_all.at[c]], add=True)

        issue(0, 0)

        def pair(c2, _):
            ce = 2 * c2
            issue(1, ce + 1)
            process(0, ce)

            @pl.when(ce + 2 < NCH)
            def _():
                issue(0, ce + 2)

            process(1, ce + 1)
            return 0

        lax.fori_loop(0, NCH // 2, pair, 0)
        plsc.subcore_barrier()
        pltpu.sync_copy(
            acc_sh.at[pl.ds(sid * ZR, ZR)], out_h.at[cid, pl.ds(sid * ZR, ZR)]
        )
        pltpu.sync_copy(
            cnt_sh.at[pl.ds(sid * ZR, ZR)], cnt_h.at[cid, pl.ds(sid * ZR, ZR)]
        )

    return k(idx_i, idx_j, pa, pb, ew)


def _finalize(parts, cnts, W2, b2):
    def body(s_ref, c_ref, w_ref, b_ref, out_ref):
        s = s_ref[0] + s_ref[1]
        c = c_ref[0] + c_ref[1]
        out_ref[...] = (
            jnp.dot(s[:NN], w_ref[...], preferred_element_type=jnp.float32)
            + c[:NN] * b_ref[...]
        )

    return pl.pallas_call(
        body,
        out_shape=jax.ShapeDtypeStruct((NN, H), jnp.float32),
    )(parts, cnts, W2, b2.reshape(1, H))


def kernel(x, edge_index, edge_attr, W1, b1, W2, b2):
    idx_i = edge_index[0].reshape(NE // CH, CH)
    idx_j = edge_index[1].reshape(NE // CH, CH)
    W1a = W1[:DF]
    W1b = W1[DF:2 * DF]
    W1c = W1[2 * DF:]
    pa, pb = _node_proj(x, W1a, W1b)
    ew = _edge_proj(edge_attr, W1c, b1)
    parts, cnts = _sc_aggregate(idx_i, idx_j, pa, pb, ew)
    cnts = cnts.reshape(NC, NP, 1)
    return _finalize(parts, cnts, W2, b2)


# speedup vs baseline: 7.3559x; 1.0220x over previous
import functools

import jax
import jax.numpy as jnp
from jax import lax
from jax.experimental import pallas as pl
from jax.experimental.pallas import tpu as pltpu
from jax.experimental.pallas import tpu_sc as plsc

NN = 10000
NE = 320000
DF = 128
DE = 16
H = 32

NC, NS, L = 2, 16, 16
NW = NC * NS
EPW = NE // NW
CH = 40
NCH = EPW // CH
NP = 10240
ZR = NP // NS


def _node_proj(x, W1a, W1b):
    def body(x_ref, wa_ref, wb_ref, pa_ref, pb_ref):
        xv = x_ref[...]
        pa_ref[...] = jnp.dot(xv, wa_ref[...], preferred_element_type=jnp.float32)
        pb_ref[...] = jnp.dot(xv, wb_ref[...], preferred_element_type=jnp.float32)

    return pl.pallas_call(
        body,
        out_shape=[jax.ShapeDtypeStruct((NN, H), jnp.float32)] * 2,
    )(x, W1a, W1b)


def _edge_proj(edge_attr, W1c, b1):
    BLK = 2000
    NB = (NE // 4) // BLK

    def body(e0_ref, e1_ref, e2_ref, e3_ref, w_ref, b_ref, out_ref):
        w = w_ref[...]
        b = b_ref[...]
        for kk, ek in enumerate((e0_ref, e1_ref, e2_ref, e3_ref)):
            out_ref[:, kk * H:(kk + 1) * H] = (
                jnp.dot(ek[...], w, preferred_element_type=jnp.float32) + b
            )

    ea_specs = [
        pl.BlockSpec((BLK, DE), lambda i, kk=kk: (kk * NB + i, 0))
        for kk in range(4)
    ]
    return pl.pallas_call(
        body,
        grid=(NB,),
        in_specs=ea_specs + [
            pl.BlockSpec((DE, H), lambda i: (0, 0)),
            pl.BlockSpec((1, H), lambda i: (0, 0)),
        ],
        out_specs=pl.BlockSpec((BLK, 4 * H), lambda i: (i, 0)),
        out_shape=jax.ShapeDtypeStruct((NE // 4, 4 * H), jnp.float32),
    )(edge_attr, edge_attr, edge_attr, edge_attr, W1c, b1.reshape(1, H))


def _sc_aggregate(idx_i, idx_j, pa, pb, ew):
    mesh = plsc.VectorSubcoreMesh(
        core_axis_name="c", subcore_axis_name="s", num_cores=NC, num_subcores=NS
    )

    @functools.partial(
        pl.kernel,
        out_type=[
            jax.ShapeDtypeStruct((NC, NP, H), jnp.float32),
            jax.ShapeDtypeStruct((NC, NP), jnp.float32),
        ],
        mesh=mesh,
        compiler_params=pltpu.CompilerParams(use_tc_tiling_on_sc=False),
        scratch_types=[
            pltpu.VMEM((NCH, CH), jnp.int32),
            pltpu.VMEM((NCH, CH), jnp.int32),
            pltpu.VMEM((2, CH, H), jnp.float32),
            pltpu.VMEM((2, CH, H), jnp.float32),
            pltpu.VMEM((2, CH, H), jnp.float32),
            pltpu.VMEM((2, CH, H), jnp.float32),
            pltpu.VMEM((CH,), jnp.float32),
            pltpu.VMEM((ZR, H), jnp.float32),
            pltpu.VMEM((ZR,), jnp.float32),
            pltpu.VMEM_SHARED((NP, H), jnp.float32),
            pltpu.VMEM_SHARED((NP,), jnp.float32),
            [pltpu.SemaphoreType.DMA] * 2,
            [pltpu.SemaphoreType.DMA] * 2,
            [pltpu.SemaphoreType.DMA] * 2,
        ],
    )
    def k(ii_h, jj_h, pa_h, pb_h, ew_h, out_h, cnt_h,
          ii_all, jj_all, pa_v, pb_v, ew_v, h_v, ones_v, z_v, zc_v,
          acc_sh, cnt_sh, sem_pa, sem_pb, sem_ew):
        cid = lax.axis_index("c")
        sid = lax.axis_index("s")
        wid = cid * NS + sid

        zeros16 = jnp.zeros((L,), jnp.float32)
        ones16 = jnp.ones((L,), jnp.float32)

        def zrow(r, _):
            z_v[r, pl.ds(0, L)] = zeros16
            z_v[r, pl.ds(L, L)] = zeros16
            return 0

        lax.fori_loop(0, ZR, zrow, 0)

        def zcrow(r, _):
            zc_v[pl.ds(r * L, L)] = zeros16
            return 0

        lax.fori_loop(0, ZR // L, zcrow, 0)

        def orow(r, _):
            ones_v[pl.ds(r * L, L)] = ones16
            return 0

        lax.fori_loop(0, max(CH // L, 1), orow, 0)

        pltpu.sync_copy(ii_h.at[pl.ds(wid * NCH, NCH)], ii_all)
        pltpu.sync_copy(jj_h.at[pl.ds(wid * NCH, NCH)], jj_all)

        pltpu.sync_copy(z_v, acc_sh.at[pl.ds(sid * ZR, ZR)])
        pltpu.sync_copy(zc_v, cnt_sh.at[pl.ds(sid * ZR, ZR)])
        plsc.subcore_barrier()

        ew_col = (wid // 8) * H
        ew_row0 = (wid % 8) * EPW

        def issue(b, c):
            pltpu.async_copy(pa_h.at[ii_all.at[c]], pa_v.at[b], sem_pa[b])
            pltpu.async_copy(pb_h.at[jj_all.at[c]], pb_v.at[b], sem_pb[b])
            pltpu.async_copy(
                ew_h.at[pl.ds(ew_row0 + c * CH, CH), pl.ds(ew_col, H)],
                ew_v.at[b], sem_ew[b],
            )

        def process(b, c):
            pltpu.make_async_copy(pa_h.at[ii_all.at[c]], pa_v.at[b], sem_pa[b]).wait()
            pltpu.make_async_copy(pb_h.at[jj_all.at[c]], pb_v.at[b], sem_pb[b]).wait()
            pltpu.make_async_copy(
                ew_h.at[pl.ds(ew_row0 + c * CH, CH), pl.ds(ew_col, H)],
                ew_v.at[b], sem_ew[b],
            ).wait()

            def edge(e, _):
                for hh in range(2):
                    a = pa_v[b, e, pl.ds(hh * L, L)]
                    bb = pb_v[b, e, pl.ds(hh * L, L)]
                    ee = ew_v[b, e, pl.ds(hh * L, L)]
                    h_v[b, e, pl.ds(hh * L, L)] = jnp.maximum(a + bb + ee, 0.0)
                return 0

            lax.fori_loop(0, CH, edge, 0)
            pltpu.sync_copy(h_v.at[b], acc_sh.at[ii_all.at[c]], add=True)
            pltpu.sync_copy(ones_v, cnt_sh.at[ii_all.at[c]], add=True)

        issue(0, 0)

        def pair(c2, _):
            ce = 2 * c2
            issue(1, ce + 1)
            process(0, ce)

            @pl.when(ce + 2 < NCH)
            def _():
                issue(0, ce + 2)

            process(1, ce + 1)
            return 0

        lax.fori_loop(0, NCH // 2, pair, 0)
        plsc.subcore_barrier()
        pltpu.sync_copy(
            acc_sh.at[pl.ds(sid * ZR, ZR)], out_h.at[cid, pl.ds(sid * ZR, ZR)]
        )
        pltpu.sync_copy(
            cnt_sh.at[pl.ds(sid * ZR, ZR)], cnt_h.at[cid, pl.ds(sid * ZR, ZR)]
        )

    return k(idx_i, idx_j, pa, pb, ew)


def _finalize(parts, cnts, W2, b2):
    def body(s_ref, c_ref, w_ref, b_ref, out_ref):
        s = s_ref[0] + s_ref[1]
        c = c_ref[0] + c_ref[1]
        out_ref[...] = (
            jnp.dot(s[:NN], w_ref[...], preferred_element_type=jnp.float32)
            + c[:NN] * b_ref[...]
        )

    return pl.pallas_call(
        body,
        out_shape=jax.ShapeDtypeStruct((NN, H), jnp.float32),
    )(parts, cnts, W2, b2.reshape(1, H))


def kernel(x, edge_index, edge_attr, W1, b1, W2, b2):
    idx_i = edge_index[0].reshape(NE // CH, CH)
    idx_j = edge_index[1].reshape(NE // CH, CH)
    W1a = W1[:DF]
    W1b = W1[DF:2 * DF]
    W1c = W1[2 * DF:]
    pa, pb = _node_proj(x, W1a, W1b)
    ew = _edge_proj(edge_attr, W1c, b1)
    parts, cnts = _sc_aggregate(idx_i, idx_j, pa, pb, ew)
    cnts = cnts.reshape(NC, NP, 1)
    return _finalize(parts, cnts, W2, b2)


# speedup vs baseline: 9.3134x; 1.2661x over previous
import functools

import jax
import jax.numpy as jnp
from jax import lax
from jax.experimental import pallas as pl
from jax.experimental.pallas import tpu as pltpu
from jax.experimental.pallas import tpu_sc as plsc

NN = 10000
NE = 320000
DF = 128
DE = 16
H = 32

NC, NS, L = 2, 16, 16
NW = NC * NS
EPW = NE // NW
CH = 40
NCH = EPW // CH
NP = 10240
ZR = NP // NS


def _node_proj(x, W1a, W1b):
    def body(x_ref, wa_ref, wb_ref, pa_ref, pb_ref):
        xv = x_ref[...]
        pa_ref[...] = jnp.dot(xv, wa_ref[...], preferred_element_type=jnp.float32)
        pb_ref[...] = jnp.dot(xv, wb_ref[...], preferred_element_type=jnp.float32)

    return pl.pallas_call(
        body,
        out_shape=[jax.ShapeDtypeStruct((NN, H), jnp.float32)] * 2,
    )(x, W1a, W1b)


def _edge_proj(edge_attr, W1c, b1):
    BLK = 3200
    NB = (NE // 4) // BLK
    dn = (((0,), (0,)), ((), ()))

    def body(e0_ref, e1_ref, e2_ref, e3_ref, w_ref, b_ref, out_ref):
        w = w_ref[...]
        b = b_ref[...]
        for kk, ek in enumerate((e0_ref, e1_ref, e2_ref, e3_ref)):
            out_ref[:, kk * H:(kk + 1) * H] = (
                lax.dot_general(ek[...], w, dn,
                                preferred_element_type=jnp.float32) + b
            )

    ea_specs = [
        pl.BlockSpec((DE, BLK), lambda i, kk=kk: (0, kk * NB + i))
        for kk in range(4)
    ]
    eaT = edge_attr.T
    return pl.pallas_call(
        body,
        grid=(NB,),
        in_specs=ea_specs + [
            pl.BlockSpec((DE, H), lambda i: (0, 0)),
            pl.BlockSpec((1, H), lambda i: (0, 0)),
        ],
        out_specs=pl.BlockSpec((BLK, 4 * H), lambda i: (i, 0)),
        out_shape=jax.ShapeDtypeStruct((NE // 4, 4 * H), jnp.float32),
    )(eaT, eaT, eaT, eaT, W1c, b1.reshape(1, H))


def _sc_aggregate(idx_i, idx_j, pa, pb, ew):
    mesh = plsc.VectorSubcoreMesh(
        core_axis_name="c", subcore_axis_name="s", num_cores=NC, num_subcores=NS
    )

    @functools.partial(
        pl.kernel,
        out_type=[
            jax.ShapeDtypeStruct((NC, NP, H), jnp.float32),
            jax.ShapeDtypeStruct((NC, NP), jnp.float32),
        ],
        mesh=mesh,
        compiler_params=pltpu.CompilerParams(use_tc_tiling_on_sc=False),
        scratch_types=[
            pltpu.VMEM((NCH, CH), jnp.int32),
            pltpu.VMEM((NCH, CH), jnp.int32),
            pltpu.VMEM((2, CH, H), jnp.float32),
            pltpu.VMEM((2, CH, H), jnp.float32),
            pltpu.VMEM((2, CH, H), jnp.float32),
            pltpu.VMEM((2, CH, H), jnp.float32),
            pltpu.VMEM((CH,), jnp.float32),
            pltpu.VMEM((ZR, H), jnp.float32),
            pltpu.VMEM((ZR,), jnp.float32),
            pltpu.VMEM_SHARED((NP, H), jnp.float32),
            pltpu.VMEM_SHARED((NP,), jnp.float32),
            [pltpu.SemaphoreType.DMA] * 2,
            [pltpu.SemaphoreType.DMA] * 2,
            [pltpu.SemaphoreType.DMA] * 2,
        ],
    )
    def k(ii_h, jj_h, pa_h, pb_h, ew_h, out_h, cnt_h,
          ii_all, jj_all, pa_v, pb_v, ew_v, h_v, ones_v, z_v, zc_v,
          acc_sh, cnt_sh, sem_pa, sem_pb, sem_ew):
        cid = lax.axis_index("c")
        sid = lax.axis_index("s")
        wid = cid * NS + sid

        zeros16 = jnp.zeros((L,), jnp.float32)
        ones16 = jnp.ones((L,), jnp.float32)

        def zrow(r, _):
            z_v[r, pl.ds(0, L)] = zeros16
            z_v[r, pl.ds(L, L)] = zeros16
            return 0

        lax.fori_loop(0, ZR, zrow, 0)

        def zcrow(r, _):
            zc_v[pl.ds(r * L, L)] = zeros16
            return 0

        lax.fori_loop(0, ZR // L, zcrow, 0)

        def orow(r, _):
            ones_v[pl.ds(r * L, L)] = ones16
            return 0

        lax.fori_loop(0, max(CH // L, 1), orow, 0)

        pltpu.sync_copy(ii_h.at[pl.ds(wid * NCH, NCH)], ii_all)
        pltpu.sync_copy(jj_h.at[pl.ds(wid * NCH, NCH)], jj_all)

        pltpu.sync_copy(z_v, acc_sh.at[pl.ds(sid * ZR, ZR)])
        pltpu.sync_copy(zc_v, cnt_sh.at[pl.ds(sid * ZR, ZR)])
        plsc.subcore_barrier()

        ew_col = (wid // 8) * H
        ew_row0 = (wid % 8) * EPW

        def issue(b, c):
            pltpu.async_copy(pa_h.at[ii_all.at[c]], pa_v.at[b], sem_pa[b])
            pltpu.async_copy(pb_h.at[jj_all.at[c]], pb_v.at[b], sem_pb[b])
            pltpu.async_copy(
                ew_h.at[pl.ds(ew_row0 + c * CH, CH), pl.ds(ew_col, H)],
                ew_v.at[b], sem_ew[b],
            )

        def process(b, c):
            pltpu.make_async_copy(pa_h.at[ii_all.at[c]], pa_v.at[b], sem_pa[b]).wait()
            pltpu.make_async_copy(pb_h.at[jj_all.at[c]], pb_v.at[b], sem_pb[b]).wait()
            pltpu.make_async_copy(
                ew_h.at[pl.ds(ew_row0 + c * CH, CH), pl.ds(ew_col, H)],
                ew_v.at[b], sem_ew[b],
            ).wait()

            def edge(e, _):
                for hh in range(2):
                    a = pa_v[b, e, pl.ds(hh * L, L)]
                    bb = pb_v[b, e, pl.ds(hh * L, L)]
                    ee = ew_v[b, e, pl.ds(hh * L, L)]
                    h_v[b, e, pl.ds(hh * L, L)] = jnp.maximum(a + bb + ee, 0.0)
                return 0

            lax.fori_loop(0, CH, edge, 0)
            pltpu.sync_copy(h_v.at[b], acc_sh.at[ii_all.at[c]], add=True)
            pltpu.sync_copy(ones_v, cnt_sh.at[ii_all.at[c]], add=True)

        issue(0, 0)

        def pair(c2, _):
            ce = 2 * c2
            issue(1, ce + 1)
            process(0, ce)

            @pl.when(ce + 2 < NCH)
            def _():
                issue(0, ce + 2)

            process(1, ce + 1)
            return 0

        lax.fori_loop(0, NCH // 2, pair, 0)
        plsc.subcore_barrier()
        pltpu.sync_copy(
            acc_sh.at[pl.ds(sid * ZR, ZR)], out_h.at[cid, pl.ds(sid * ZR, ZR)]
        )
        pltpu.sync_copy(
            cnt_sh.at[pl.ds(sid * ZR, ZR)], cnt_h.at[cid, pl.ds(sid * ZR, ZR)]
        )

    return k(idx_i, idx_j, pa, pb, ew)


def _finalize(parts, cnts, W2, b2):
    def body(s_ref, c_ref, w_ref, b_ref, out_ref):
        s = s_ref[0] + s_ref[1]
        c = c_ref[0] + c_ref[1]
        out_ref[...] = (
            jnp.dot(s[:NN], w_ref[...], preferred_element_type=jnp.float32)
            + c[:NN] * b_ref[...]
        )

    return pl.pallas_call(
        body,
        out_shape=jax.ShapeDtypeStruct((NN, H), jnp.float32),
    )(parts, cnts, W2, b2.reshape(1, H))


def kernel(x, edge_index, edge_attr, W1, b1, W2, b2):
    idx_i = edge_index[0].reshape(NE // CH, CH)
    idx_j = edge_index[1].reshape(NE // CH, CH)
    W1a = W1[:DF]
    W1b = W1[DF:2 * DF]
    W1c = W1[2 * DF:]
    pa, pb = _node_proj(x, W1a, W1b)
    ew = _edge_proj(edge_attr, W1c, b1)
    parts, cnts = _sc_aggregate(idx_i, idx_j, pa, pb, ew)
    cnts = cnts.reshape(NC, NP, 1)
    return _finalize(parts, cnts, W2, b2)
